# Initial kernel scaffold; baseline (speedup 1.0000x reference)
import jax
import jax.numpy as jnp
from jax.experimental import pallas as pl


def kernel(h, lengths):
    raise NotImplementedError("write your pallas kernel here")


# speedup vs baseline: 3.2089x; 3.2089x over previous
import jax
import jax.numpy as jnp
from jax.experimental import pallas as pl


def _body(nl_ref, in0_ref, in1_ref, out_ref, *, TB, B, D):
    i = pl.program_id(0)
    t = i * TB + jax.lax.broadcasted_iota(jnp.int32, (TB, B, D), 0)
    mask = t < nl_ref[...]
    out_ref[:, :, :D] = jnp.where(mask, in0_ref[:, 0, :, :], 0.0)
    out_ref[:, :, D:] = jnp.where(mask, in1_ref[:, 0, :, :], 0.0)


def kernel(h, lengths):
    T, B, D = h.shape
    TH = T // 2
    new_len = (lengths // 2).astype(jnp.int32)
    nl2 = jnp.broadcast_to(new_len[None, :, None], (1, B, D))
    h4 = h.reshape(TH, 2, B, D)
    TB = 32
    import functools
    body = functools.partial(_body, TB=TB, B=B, D=D)
    h_cat = pl.pallas_call(
        body,
        grid=(TH // TB,),
        in_specs=[
            pl.BlockSpec((1, B, D), lambda i: (0, 0, 0)),
            pl.BlockSpec((TB, 1, B, D), lambda i: (i, 0, 0, 0)),
            pl.BlockSpec((TB, 1, B, D), lambda i: (i, 1, 0, 0)),
        ],
        out_specs=pl.BlockSpec((TB, B, 2 * D), lambda i: (i, 0, 0)),
        out_shape=jax.ShapeDtypeStruct((TH, B, 2 * D), h.dtype),
    )(nl2, h4, h4)
    return h_cat, new_len
